# baseline (speedup 1.0000x reference)
import dataclasses
import functools

import jax
import jax.numpy as jnp
from jax import lax
from jax.experimental import pallas as pl
from jax.experimental.pallas import tpu as pltpu
from jax.experimental.pallas import tpu_sc as plsc

_NC = 2
_NS = 16
_NW = _NC * _NS
_L = 16


def _node_score_body(h_ref, w_ref, b_ref, p_ref):
    x = h_ref[...]
    R, D = x.shape
    w2 = jnp.concatenate([w_ref[:, :D], w_ref[:, D:]], axis=0)
    for s in range(R // 128):
        a = x[s * 128:(s + 1) * 128, :]
        r = lax.dot_general(w2, a, (((1,), (1,)), ((), ())),
                            precision=lax.Precision.HIGHEST,
                            preferred_element_type=jnp.float32)
        p_ref[0, s] = r[0] + b_ref[0]
        p_ref[1, s] = r[1]


@functools.lru_cache(maxsize=None)
def _node_score_call(N, D, R):
    G = pl.cdiv(N, R)
    NP = G * R // 128
    return pl.pallas_call(
        _node_score_body,
        grid=(G,),
        in_specs=[
            pl.BlockSpec((R, D), lambda i: (i, 0)),
            pl.BlockSpec((1, 2 * D), lambda i: (0, 0)),
            pl.BlockSpec(memory_space=pltpu.SMEM),
        ],
        out_specs=pl.BlockSpec((2, R // 128, 128), lambda i: (0, i, 0)),
        out_shape=jax.ShapeDtypeStruct((2, NP, 128), jnp.float32),
    )


@functools.lru_cache(maxsize=None)
def _edge_score_call(E, NP):
    CH = 128 * ((E // _NW + 127) // 128)
    assert CH % 64 == 0 and (E - CH) % 128 == 0 and E >= CH
    mesh = plsc.VectorSubcoreMesh(core_axis_name="c", subcore_axis_name="s")
    cp = pltpu.CompilerParams()
    if "needs_layout_passes" in pltpu.CompilerParams.__dataclass_fields__:
        cp = dataclasses.replace(cp, needs_layout_passes=False)

    @functools.partial(
        pl.kernel,
        compiler_params=cp,
        out_type=jax.ShapeDtypeStruct((E,), jnp.float32),
        mesh=mesh,
        scratch_types=[
            pltpu.VMEM((2, CH), jnp.int32),
            pltpu.VMEM((2, NP, 128), jnp.float32),
            pltpu.VMEM((CH,), jnp.float32),
            pltpu.SemaphoreType.DMA,
        ],
    )
    def edge_kernel(p_hbm, ei_hbm, out_hbm, ei_v, p_v, out_v, sem):
        wid = lax.axis_index("s") * _NC + lax.axis_index("c")
        base = jnp.minimum(wid * CH, E - CH)

        c1 = pltpu.async_copy(p_hbm, p_v, sem)
        c2 = pltpu.async_copy(ei_hbm.at[:, pl.ds(base, CH)], ei_v, sem)
        c1.wait()
        c2.wait()

        zero = jnp.zeros((_L,), jnp.int32)
        one = jnp.ones((_L,), jnp.int32)
        mask = jnp.full((_L,), 127, jnp.int32)

        @plsc.parallel_loop(0, CH, step=_L, unroll=8)
        def _(o):
            s = ei_v[0, pl.ds(o, _L)]
            d = ei_v[1, pl.ds(o, _L)]
            gu = plsc.load_gather(
                p_v, [zero, lax.shift_right_logical(s, 7), s & mask])
            gv = plsc.load_gather(
                p_v, [one, lax.shift_right_logical(d, 7), d & mask])
            out_v[pl.ds(o, _L)] = gu + gv

        pltpu.sync_copy(out_v, out_hbm.at[pl.ds(base, CH)])

    return edge_kernel


def kernel(h, edge_index, W, b):
    N, D = h.shape
    E = edge_index.shape[1]
    p = _node_score_call(N, D, 2048)(h, W, b)
    score = _edge_score_call(E, p.shape[1])(p, edge_index)
    return score.reshape(E, 1)


# speedup vs baseline: 1.1265x; 1.1265x over previous
import dataclasses
import functools

import jax
import jax.numpy as jnp
from jax import lax
from jax.experimental import pallas as pl
from jax.experimental.pallas import tpu as pltpu
from jax.experimental.pallas import tpu_sc as plsc

_NC = 2
_NS = 16
_NW = _NC * _NS
_L = 16


def _node_score_body(h_ref, w_ref, b_ref, p_ref):
    x = h_ref[...]
    R, D = x.shape
    su = jnp.sum(x * w_ref[:, :D], axis=1) + b_ref[0]
    sv = jnp.sum(x * w_ref[:, D:], axis=1)
    p_ref[0] = su.reshape(R // 128, 128)
    p_ref[1] = sv.reshape(R // 128, 128)


@functools.lru_cache(maxsize=None)
def _node_score_call(N, D, R):
    G = pl.cdiv(N, R)
    NP = G * R // 128
    return pl.pallas_call(
        _node_score_body,
        grid=(G,),
        in_specs=[
            pl.BlockSpec((R, D), lambda i: (i, 0)),
            pl.BlockSpec((1, 2 * D), lambda i: (0, 0)),
            pl.BlockSpec(memory_space=pltpu.SMEM),
        ],
        out_specs=pl.BlockSpec((2, R // 128, 128), lambda i: (0, i, 0)),
        out_shape=jax.ShapeDtypeStruct((2, NP, 128), jnp.float32),
    )


@functools.lru_cache(maxsize=None)
def _edge_score_call(E, NP):
    CH = 128 * ((E // _NW + 127) // 128)
    assert CH % 64 == 0 and (E - CH) % 128 == 0 and E >= CH
    mesh = plsc.VectorSubcoreMesh(core_axis_name="c", subcore_axis_name="s")
    cp = pltpu.CompilerParams()
    if "needs_layout_passes" in pltpu.CompilerParams.__dataclass_fields__:
        cp = dataclasses.replace(cp, needs_layout_passes=False)

    @functools.partial(
        pl.kernel,
        compiler_params=cp,
        out_type=jax.ShapeDtypeStruct((E,), jnp.float32),
        mesh=mesh,
        scratch_types=[
            pltpu.VMEM((2, CH), jnp.int32),
            pltpu.VMEM((2, NP, 128), jnp.float32),
            pltpu.VMEM((CH,), jnp.float32),
            pltpu.SemaphoreType.DMA,
        ],
    )
    def edge_kernel(p_hbm, ei_hbm, out_hbm, ei_v, p_v, out_v, sem):
        wid = lax.axis_index("s") * _NC + lax.axis_index("c")
        base = jnp.minimum(wid * CH, E - CH)

        c1 = pltpu.async_copy(p_hbm, p_v, sem)
        c2 = pltpu.async_copy(ei_hbm.at[:, pl.ds(base, CH)], ei_v, sem)
        c1.wait()
        c2.wait()

        zero = jnp.zeros((_L,), jnp.int32)
        one = jnp.ones((_L,), jnp.int32)
        mask = jnp.full((_L,), 127, jnp.int32)

        @plsc.parallel_loop(0, CH, step=_L, unroll=8)
        def _(o):
            s = ei_v[0, pl.ds(o, _L)]
            d = ei_v[1, pl.ds(o, _L)]
            gu = plsc.load_gather(
                p_v, [zero, lax.shift_right_logical(s, 7), s & mask])
            gv = plsc.load_gather(
                p_v, [one, lax.shift_right_logical(d, 7), d & mask])
            out_v[pl.ds(o, _L)] = gu + gv

        pltpu.sync_copy(out_v, out_hbm.at[pl.ds(base, CH)])

    return edge_kernel


def kernel(h, edge_index, W, b):
    N, D = h.shape
    E = edge_index.shape[1]
    p = _node_score_call(N, D, 2048)(h, W, b)
    score = _edge_score_call(E, p.shape[1])(p, edge_index)
    return score.reshape(E, 1)
